# baseline (speedup 1.0000x reference)
import functools

import jax
import jax.numpy as jnp
from jax import lax
from jax.experimental import pallas as pl
from jax.experimental.pallas import tpu as pltpu
from jax.experimental.pallas import tpu_sc as plsc

N = 100000
DIM = 128
ATOM_DIM = 6
MAX_DIS = 10
LANES = 16
NSEG = DIM // LANES

CHUNK = 400
NCHUNKS = N // CHUNK
NWORKERS = 32
CHUNKS_PER_WORKER = (NCHUNKS + NWORKERS - 1) // NWORKERS


def _toff(dvec):
    d_i = dvec[0]
    dc = jnp.where(d_i > 1000, MAX_DIS + 1, jnp.minimum(d_i, MAX_DIS))
    return dc * DIM


def _body(xf_hbm, d_hbm, w_hbm, b_hbm, embf_hbm, outf_hbm,
          xf_v, d_v, w_v, b_v, t2f_v, out0_v, out1_v, sem0, sem1):
    wid = lax.axis_index("c") * 16 + lax.axis_index("s")

    pltpu.sync_copy(w_hbm, w_v)
    pltpu.sync_copy(b_hbm, b_v)
    pltpu.sync_copy(embf_hbm, t2f_v)
    for r in range(MAX_DIS + 2):
        for s in range(NSEG):
            sl = pl.ds(r * DIM + s * LANES, LANES)
            t2f_v[sl] = t2f_v[sl] + b_v[pl.ds(s * LANES, LANES)]

    out_bufs = (out0_v, out1_v)
    sems = (sem0, sem1)
    copies = [None, None]

    for t in range(CHUNKS_PER_WORKER):
        k = jnp.minimum(wid + t * NWORKERS, NCHUNKS - 1)
        base = k * CHUNK
        buf = t % 2
        out_v = out_bufs[buf]

        pltpu.sync_copy(xf_hbm.at[pl.ds(base * ATOM_DIM, CHUNK * ATOM_DIM)],
                        xf_v.at[pl.ds(0, CHUNK * ATOM_DIM)])
        pltpu.sync_copy(d_hbm.at[pl.ds(base, CHUNK)],
                        d_v.at[pl.ds(0, CHUNK)])
        if copies[buf] is not None:
            copies[buf].wait()

        for segs in (range(0, NSEG // 2), range(NSEG // 2, NSEG)):
            segs = list(segs)
            w_regs = [[w_v[j, pl.ds(s * LANES, LANES)] for s in segs]
                      for j in range(ATOM_DIM)]

            def row(i, toff, segs=segs, w_regs=w_regs, out_v=out_v):
                toff_next = _toff(d_v[pl.ds(i + 1, LANES)])
                xvec = xf_v[pl.ds(i * ATOM_DIM, LANES)]
                obase = i * DIM
                for si, s in enumerate(segs):
                    t2seg = t2f_v[pl.ds(toff + s * LANES, LANES)]
                    p = [xvec[j] * w_regs[j][si] for j in range(ATOM_DIM)]
                    acc = (((t2seg + p[0]) + (p[1] + p[2]))
                           + ((p[3] + p[4]) + p[5]))
                    out_v[pl.ds(obase + s * LANES, LANES)] = acc
                return toff_next

            lax.fori_loop(0, CHUNK, row, _toff(d_v[pl.ds(0, LANES)]))
        copies[buf] = pltpu.async_copy(
            out_v, outf_hbm.at[pl.ds(base * DIM, CHUNK * DIM)], sems[buf])

    for c in copies:
        c.wait()


@jax.jit
def _run(xf, d, W, b, embf):
    mesh = plsc.VectorSubcoreMesh(core_axis_name="c", subcore_axis_name="s")
    kern = functools.partial(
        pl.kernel,
        mesh=mesh,
        out_type=jax.ShapeDtypeStruct((N * DIM,), jnp.float32),
        scratch_types=[
            pltpu.VMEM((CHUNK * ATOM_DIM + LANES,), jnp.float32),
            pltpu.VMEM((CHUNK + LANES,), jnp.int32),
            pltpu.VMEM((ATOM_DIM, DIM), jnp.float32),
            pltpu.VMEM((DIM,), jnp.float32),
            pltpu.VMEM(((MAX_DIS + 2) * DIM,), jnp.float32),
            pltpu.VMEM((CHUNK * DIM,), jnp.float32),
            pltpu.VMEM((CHUNK * DIM,), jnp.float32),
            pltpu.SemaphoreType.DMA,
            pltpu.SemaphoreType.DMA,
        ],
    )(_body)
    return kern(xf, d, W, b, embf)


def kernel(x, d, W, b, emb_d):
    out = _run(x.reshape(-1), d, W, b, emb_d.reshape(-1))
    return out.reshape(N, DIM)


# speedup vs baseline: 1.0404x; 1.0404x over previous
import functools

import jax
import jax.numpy as jnp
from jax import lax
from jax.experimental import pallas as pl
from jax.experimental.pallas import tpu as pltpu
from jax.experimental.pallas import tpu_sc as plsc

N = 100000
DIM = 128
ATOM_DIM = 6
MAX_DIS = 10
LANES = 16
NSEG = DIM // LANES

CHUNK = 400
NCHUNKS = N // CHUNK
NWORKERS = 32
CHUNKS_PER_WORKER = (NCHUNKS + NWORKERS - 1) // NWORKERS


def _toff(dvec):
    d_i = dvec[0]
    dc = jnp.where(d_i > 1000, MAX_DIS + 1, jnp.minimum(d_i, MAX_DIS))
    return dc * DIM


def _body(xf_hbm, d_hbm, w_hbm, b_hbm, embf_hbm, out_hbm,
          xf_v, d_v, w_v, b_v, t2f_v, out0_v, out1_v, sem0, sem1):
    wid = lax.axis_index("c") * 16 + lax.axis_index("s")

    pltpu.sync_copy(w_hbm, w_v)
    pltpu.sync_copy(b_hbm, b_v)
    pltpu.sync_copy(embf_hbm, t2f_v)
    for r in range(MAX_DIS + 2):
        for s in range(NSEG):
            sl = pl.ds(r * DIM + s * LANES, LANES)
            t2f_v[sl] = t2f_v[sl] + b_v[pl.ds(s * LANES, LANES)]

    w_regs = [[w_v[j, pl.ds(s * LANES, LANES)] for s in range(NSEG)]
              for j in range(ATOM_DIM)]

    out_bufs = (out0_v, out1_v)
    sems = (sem0, sem1)
    copies = [None, None]

    for t in range(CHUNKS_PER_WORKER):
        k = jnp.minimum(wid + t * NWORKERS, NCHUNKS - 1)
        base = k * CHUNK
        buf = t % 2
        out_v = out_bufs[buf]

        pltpu.sync_copy(xf_hbm.at[pl.ds(base * ATOM_DIM, CHUNK * ATOM_DIM)],
                        xf_v.at[pl.ds(0, CHUNK * ATOM_DIM)])
        pltpu.sync_copy(d_hbm.at[pl.ds(base, CHUNK)],
                        d_v.at[pl.ds(0, CHUNK)])
        if copies[buf] is not None:
            copies[buf].wait()

        def row(i, toff, out_v=out_v):
            toff_next = _toff(d_v[pl.ds(i + 1, LANES)])
            xvec = xf_v[pl.ds(i * ATOM_DIM, LANES)]
            for s in range(NSEG):
                t2seg = t2f_v[pl.ds(toff + s * LANES, LANES)]
                p = [xvec[j] * w_regs[j][s] for j in range(ATOM_DIM)]
                acc = ((t2seg + p[0]) + (p[1] + p[2])) + ((p[3] + p[4]) + p[5])
                out_v[i, pl.ds(s * LANES, LANES)] = acc
            return toff_next

        lax.fori_loop(0, CHUNK, row, _toff(d_v[pl.ds(0, LANES)]))
        copies[buf] = pltpu.async_copy(
            out_v, out_hbm.at[pl.ds(base, CHUNK)], sems[buf])

    for c in copies:
        c.wait()


@jax.jit
def _run(xf, d, W, b, embf):
    mesh = plsc.VectorSubcoreMesh(core_axis_name="c", subcore_axis_name="s")
    kern = functools.partial(
        pl.kernel,
        mesh=mesh,
        out_type=jax.ShapeDtypeStruct((N, DIM), jnp.float32),
        scratch_types=[
            pltpu.VMEM((CHUNK * ATOM_DIM + LANES,), jnp.float32),
            pltpu.VMEM((CHUNK + LANES,), jnp.int32),
            pltpu.VMEM((ATOM_DIM, DIM), jnp.float32),
            pltpu.VMEM((DIM,), jnp.float32),
            pltpu.VMEM(((MAX_DIS + 2) * DIM,), jnp.float32),
            pltpu.VMEM((CHUNK, DIM), jnp.float32),
            pltpu.VMEM((CHUNK, DIM), jnp.float32),
            pltpu.SemaphoreType.DMA,
            pltpu.SemaphoreType.DMA,
        ],
    )(_body)
    return kern(xf, d, W, b, embf)


def kernel(x, d, W, b, emb_d):
    return _run(x.reshape(-1), d, W, b, emb_d.reshape(-1))


# speedup vs baseline: 1.6967x; 1.6308x over previous
import functools

import jax
import jax.numpy as jnp
from jax import lax
from jax.experimental import pallas as pl
from jax.experimental.pallas import tpu as pltpu
from jax.experimental.pallas import tpu_sc as plsc

N = 100000
DIM = 128
ATOM_DIM = 6
MAX_DIS = 10
LANES = 16
NSEG = DIM // LANES

CHUNK = 400
NCHUNKS = N // CHUNK
NWORKERS = 32
CHUNKS_PER_WORKER = (NCHUNKS + NWORKERS - 1) // NWORKERS

TCB = 2000


def _toff(dvec):
    d_i = dvec[0]
    dc = jnp.where(d_i > 1000, MAX_DIS + 1, jnp.minimum(d_i, MAX_DIS))
    return dc * DIM


def _sc_body(d_hbm, b_hbm, embf_hbm, de_hbm,
             d_v, b_v, t2f_v, out0_v, out1_v, sem0, sem1):
    wid = lax.axis_index("c") * 16 + lax.axis_index("s")

    pltpu.sync_copy(b_hbm, b_v)
    pltpu.sync_copy(embf_hbm, t2f_v)
    for r in range(MAX_DIS + 2):
        for s in range(NSEG):
            sl = pl.ds(r * DIM + s * LANES, LANES)
            t2f_v[sl] = t2f_v[sl] + b_v[pl.ds(s * LANES, LANES)]

    out_bufs = (out0_v, out1_v)
    sems = (sem0, sem1)
    copies = [None, None]

    for t in range(CHUNKS_PER_WORKER):
        k = jnp.minimum(wid + t * NWORKERS, NCHUNKS - 1)
        base = k * CHUNK
        buf = t % 2
        out_v = out_bufs[buf]

        pltpu.sync_copy(d_hbm.at[pl.ds(base, CHUNK)],
                        d_v.at[pl.ds(0, CHUNK)])
        if copies[buf] is not None:
            copies[buf].wait()

        def row(i, toff, out_v=out_v):
            toff_next = _toff(d_v[pl.ds(i + 1, LANES)])
            for s in range(NSEG):
                out_v[i, pl.ds(s * LANES, LANES)] = \
                    t2f_v[pl.ds(toff + s * LANES, LANES)]
            return toff_next

        lax.fori_loop(0, CHUNK, row, _toff(d_v[pl.ds(0, LANES)]))
        copies[buf] = pltpu.async_copy(
            out_v, de_hbm.at[pl.ds(base, CHUNK)], sems[buf])

    for c in copies:
        c.wait()


def _tc_body(x_ref, de_ref, w_ref, out_ref):
    out_ref[...] = (
        jnp.dot(x_ref[...], w_ref[...], preferred_element_type=jnp.float32)
        + de_ref[...])


@jax.jit
def _run(x, d, W, b, embf):
    mesh = plsc.VectorSubcoreMesh(core_axis_name="c", subcore_axis_name="s")
    sc_kern = functools.partial(
        pl.kernel,
        mesh=mesh,
        out_type=jax.ShapeDtypeStruct((N, DIM), jnp.float32),
        scratch_types=[
            pltpu.VMEM((CHUNK + LANES,), jnp.int32),
            pltpu.VMEM((DIM,), jnp.float32),
            pltpu.VMEM(((MAX_DIS + 2) * DIM,), jnp.float32),
            pltpu.VMEM((CHUNK, DIM), jnp.float32),
            pltpu.VMEM((CHUNK, DIM), jnp.float32),
            pltpu.SemaphoreType.DMA,
            pltpu.SemaphoreType.DMA,
        ],
    )(_sc_body)
    de = sc_kern(d, b, embf)

    return pl.pallas_call(
        _tc_body,
        grid=(N // TCB,),
        in_specs=[
            pl.BlockSpec((TCB, ATOM_DIM), lambda i: (i, 0)),
            pl.BlockSpec((TCB, DIM), lambda i: (i, 0)),
            pl.BlockSpec((ATOM_DIM, DIM), lambda i: (0, 0)),
        ],
        out_specs=pl.BlockSpec((TCB, DIM), lambda i: (i, 0)),
        out_shape=jax.ShapeDtypeStruct((N, DIM), jnp.float32),
    )(x, de, W)


def kernel(x, d, W, b, emb_d):
    return _run(x, d, W, b, emb_d.reshape(-1))
